# baseline (speedup 1.0000x reference)
import jax
import jax.numpy as jnp
from jax import lax
from jax.experimental import pallas as pl
from jax.experimental.pallas import tpu as pltpu
from jax.experimental.pallas import tpu_sc as plsc

_B = 16
_W = 512
_M = _W * _W
_K = int(0.8 * _M)
_RPT = 256
_RPC = 32
_NCH = _RPT // _RPC
_SHIFT = 17
_NB = 1 << 14
_MID = 1 << (_SHIFT - 1)
_MW = 2048


def _centers(o):
    bits = ((o + jnp.arange(16, dtype=jnp.int32)) << _SHIFT) | _MID
    return lax.bitcast_convert_type(bits, jnp.float32)


def _sc_body(pred, targ, out, pb0, pb1, tb0, tb1, ctmp, hcnt,
             sbc, sbs, obuf, shc, psem0, psem1, tsem0, tsem1):
    c = lax.axis_index("c")
    s = lax.axis_index("s")
    img = c * 8 + s // 2
    half = s % 2
    slot = s // 2
    row0 = half * _RPT

    zc = jnp.zeros((16,), jnp.int32)
    zf = jnp.zeros((16,), jnp.float32)
    ones = jnp.ones((16,), jnp.int32)

    def start(g, pb, tb, ps, ts):
        pltpu.async_copy(pred.at[img, pl.ds(row0 + g * _RPC, _RPC), :], pb, ps)
        pltpu.async_copy(targ.at[img, pl.ds(row0 + g * _RPC, _RPC), :], tb, ts)

    def wait(pb, tb, ps, ts):
        pltpu.make_async_copy(pred.at[img, pl.ds(row0, _RPC), :], pb, ps).wait()
        pltpu.make_async_copy(targ.at[img, pl.ds(row0, _RPC), :], tb, ts).wait()

    def process(pb, tb):
        @plsc.parallel_loop(0, _W // 16, unroll=3)
        def _proc(k):
            for r in range(_RPC):
                p = pb[r, pl.ds(k * 16, 16)]
                t = tb[r, pl.ds(k * 16, 16)]
                x = jnp.abs(p - t)
                b = lax.shift_right_logical(
                    lax.bitcast_convert_type(x, jnp.int32), _SHIFT)
                plsc.addupdate_scatter(hcnt, [b], ones)

    start(0, pb0, tb0, psem0, tsem0)
    start(1, pb1, tb1, psem1, tsem1)

    @plsc.parallel_loop(0, _NB // 16, unroll=4)
    def _zero(i):
        hcnt[pl.ds(i * 16, 16)] = zc

    def gbody(h, _):
        g = h * 2
        wait(pb0, tb0, psem0, tsem0)

        @pl.when(g + 2 < _NCH)
        def _():
            start(g + 2, pb0, tb0, psem0, tsem0)

        process(pb0, tb0)
        wait(pb1, tb1, psem1, tsem1)

        @pl.when(g + 3 < _NCH)
        def _():
            start(g + 3, pb1, tb1, psem1, tsem1)

        process(pb1, tb1)
        return 0

    lax.fori_loop(0, _NCH // 2, gbody, 0)

    @pl.when(half == 1)
    def _publish():
        pltpu.sync_copy(hcnt, shc.at[slot])

    plsc.subcore_barrier()

    @pl.when(half == 0)
    def _scan():
        for kb in range(_NB // _MW):
            pltpu.sync_copy(shc.at[slot, pl.ds(kb * _MW, _MW)], ctmp)

            @plsc.parallel_loop(0, _MW // 16, unroll=4)
            def _merge(i):
                o = kb * _MW + i * 16
                hcnt[pl.ds(o, 16)] = hcnt[pl.ds(o, 16)] + ctmp[pl.ds(i * 16, 16)]

        @plsc.parallel_loop(0, _NB // 256)
        def _sblk(sb):
            def inner(t, acc):
                o = sb * 256 + t * 16
                cv = hcnt[pl.ds(o, 16)]
                return (acc[0] + cv,
                        acc[1] + cv.astype(jnp.float32) * _centers(o))
            accc, accs = lax.fori_loop(0, 16, inner, (zc, zf), unroll=4)
            sbc[sb] = jnp.sum(accc)
            sbs[sb] = jnp.sum(accs)

        def bbody(j, carry):
            cnt_so, sum_so, sb_star, found = carry
            new = cnt_so + sbc[j]
            cross = jnp.logical_and(found == 0, new >= _K)
            sb_star = jnp.where(cross, j, sb_star)
            found = jnp.where(cross, jnp.int32(1), found)
            take = found == 0
            cnt_so = jnp.where(take, new, cnt_so)
            sum_so = jnp.where(take, sum_so + sbs[j], sum_so)
            return cnt_so, sum_so, sb_star, found

        cnt_so, sum_so, sb_star, _f = lax.fori_loop(
            0, _NB // 256, bbody,
            (jnp.int32(0), jnp.float32(0.0), jnp.int32(0), jnp.int32(0)))

        def cbody(t, carry):
            cnt_so, sum_so, b_star, found = carry
            o = sb_star * 256 + t * 16
            cv = hcnt[pl.ds(o, 16)]
            new = cnt_so + jnp.sum(cv)
            cross = jnp.logical_and(found == 0, new >= _K)
            b_star = jnp.where(cross, t, b_star)
            found = jnp.where(cross, jnp.int32(1), found)
            take = found == 0
            cnt_so = jnp.where(take, new, cnt_so)
            sum_so = jnp.where(
                take,
                sum_so + jnp.sum(cv.astype(jnp.float32) * _centers(o)),
                sum_so)
            return cnt_so, sum_so, b_star, found

        cnt_so2, sum_so2, b_star, _f2 = lax.fori_loop(
            0, 16, cbody, (cnt_so, sum_so, jnp.int32(0), jnp.int32(0)))

        o = sb_star * 256 + b_star * 16
        cv = hcnt[pl.ds(o, 16)]
        ctr = _centers(o)
        cum = plsc.cumsum(cv) + cnt_so2
        below = cum < _K
        prefix = cum - cv
        onehot = jnp.logical_and(jnp.logical_not(below), prefix < _K)
        cnt_below = cnt_so2 + jnp.sum(jnp.where(below, cv, 0))
        sum_below = sum_so2 + jnp.sum(
            jnp.where(below, cv.astype(jnp.float32) * ctr, zf))
        ctr_bkt = jnp.sum(jnp.where(onehot, ctr, zf))
        r = (_K - cnt_below).astype(jnp.float32)
        obuf[...] = (jnp.full((16,), sum_below, jnp.float32)
                     + jnp.full((16,), r, jnp.float32)
                     * jnp.full((16,), ctr_bkt, jnp.float32))
        pltpu.sync_copy(obuf, out.at[pl.ds(img * 16, 16)])


def kernel(prediction, target, mask):
    p = prediction.reshape(_B, _W, _W)
    t = target.reshape(_B, _W, _W)
    mesh = plsc.VectorSubcoreMesh(core_axis_name="c", subcore_axis_name="s",
                                  num_cores=2, num_subcores=16)
    sums = pl.kernel(
        _sc_body,
        out_type=jax.ShapeDtypeStruct((_B * 16,), jnp.float32),
        mesh=mesh,
        compiler_params=pltpu.CompilerParams(needs_layout_passes=False),
        scratch_types=[
            pltpu.VMEM((_RPC, _W), jnp.float32),
            pltpu.VMEM((_RPC, _W), jnp.float32),
            pltpu.VMEM((_RPC, _W), jnp.float32),
            pltpu.VMEM((_RPC, _W), jnp.float32),
            pltpu.VMEM((_MW,), jnp.int32),
            pltpu.VMEM((_NB,), jnp.int32),
            pltpu.SMEM((_NB // 256,), jnp.int32),
            pltpu.SMEM((_NB // 256,), jnp.float32),
            pltpu.VMEM((16,), jnp.float32),
            pltpu.VMEM_SHARED((8, _NB), jnp.int32),
            pltpu.SemaphoreType.DMA,
            pltpu.SemaphoreType.DMA,
            pltpu.SemaphoreType.DMA,
            pltpu.SemaphoreType.DMA,
        ],
    )(p, t)
    return jnp.mean(sums.reshape(_B, 16)[:, 0]) / (2.0 * _M)


# speedup vs baseline: 1.1118x; 1.1118x over previous
import jax
import jax.numpy as jnp
from jax import lax
from jax.experimental import pallas as pl
from jax.experimental.pallas import tpu as pltpu
from jax.experimental.pallas import tpu_sc as plsc

_B = 16
_W = 512
_M = _W * _W
_K = int(0.8 * _M)
_RPT = 256
_RPC = 32
_NCH = _RPT // _RPC
_SHIFT = 17
_NB = 1 << 14
_MID = 1 << (_SHIFT - 1)
_MW = 2048


def _centers(o):
    bits = ((o + jnp.arange(16, dtype=jnp.int32)) << _SHIFT) | _MID
    return lax.bitcast_convert_type(bits, jnp.float32)


def _sc_body(pred, targ, out, pb0, pb1, tb0, tb1, ctmp, hcnt,
             sbc, sbs, obuf, shc, psem0, psem1, tsem0, tsem1):
    c = lax.axis_index("c")
    s = lax.axis_index("s")
    img = c * 8 + s // 2
    half = s % 2
    slot = s // 2
    row0 = half * _RPT

    zc = jnp.zeros((16,), jnp.int32)
    zf = jnp.zeros((16,), jnp.float32)
    ones = jnp.ones((16,), jnp.int32)

    def start(g, pb, tb, ps, ts):
        pltpu.async_copy(pred.at[img, pl.ds(row0 + g * _RPC, _RPC), :], pb, ps)
        pltpu.async_copy(targ.at[img, pl.ds(row0 + g * _RPC, _RPC), :], tb, ts)

    def wait(pb, tb, ps, ts):
        pltpu.make_async_copy(pred.at[img, pl.ds(row0, _RPC), :], pb, ps).wait()
        pltpu.make_async_copy(targ.at[img, pl.ds(row0, _RPC), :], tb, ts).wait()

    def process(pb, tb):
        @plsc.parallel_loop(0, _W // 16, unroll=1)
        def _proc(k):
            for r in range(_RPC):
                p = pb[r, pl.ds(k * 16, 16)]
                t = tb[r, pl.ds(k * 16, 16)]
                x = jnp.abs(p - t)
                b = lax.shift_right_logical(
                    lax.bitcast_convert_type(x, jnp.int32), _SHIFT)
                plsc.addupdate_scatter(hcnt, [b], ones)

    start(0, pb0, tb0, psem0, tsem0)
    start(1, pb1, tb1, psem1, tsem1)

    @plsc.parallel_loop(0, _NB // 16, unroll=4)
    def _zero(i):
        hcnt[pl.ds(i * 16, 16)] = zc

    def gbody(h, _):
        g = h * 2
        wait(pb0, tb0, psem0, tsem0)

        @pl.when(g + 2 < _NCH)
        def _():
            start(g + 2, pb0, tb0, psem0, tsem0)

        process(pb0, tb0)
        wait(pb1, tb1, psem1, tsem1)

        @pl.when(g + 3 < _NCH)
        def _():
            start(g + 3, pb1, tb1, psem1, tsem1)

        process(pb1, tb1)
        return 0

    lax.fori_loop(0, _NCH // 2, gbody, 0)

    @pl.when(half == 1)
    def _publish():
        pltpu.sync_copy(hcnt, shc.at[slot])

    plsc.subcore_barrier()

    @pl.when(half == 0)
    def _scan():
        for kb in range(_NB // _MW):
            pltpu.sync_copy(shc.at[slot, pl.ds(kb * _MW, _MW)], ctmp)

            @plsc.parallel_loop(0, _MW // 16, unroll=4)
            def _merge(i):
                o = kb * _MW + i * 16
                hcnt[pl.ds(o, 16)] = hcnt[pl.ds(o, 16)] + ctmp[pl.ds(i * 16, 16)]

        @plsc.parallel_loop(0, _NB // 256)
        def _sblk(sb):
            def inner(t, acc):
                o = sb * 256 + t * 16
                cv = hcnt[pl.ds(o, 16)]
                return (acc[0] + cv,
                        acc[1] + cv.astype(jnp.float32) * _centers(o))
            accc, accs = lax.fori_loop(0, 16, inner, (zc, zf), unroll=4)
            sbc[sb] = jnp.sum(accc)
            sbs[sb] = jnp.sum(accs)

        def bbody(j, carry):
            cnt_so, sum_so, sb_star, found = carry
            new = cnt_so + sbc[j]
            cross = jnp.logical_and(found == 0, new >= _K)
            sb_star = jnp.where(cross, j, sb_star)
            found = jnp.where(cross, jnp.int32(1), found)
            take = found == 0
            cnt_so = jnp.where(take, new, cnt_so)
            sum_so = jnp.where(take, sum_so + sbs[j], sum_so)
            return cnt_so, sum_so, sb_star, found

        cnt_so, sum_so, sb_star, _f = lax.fori_loop(
            0, _NB // 256, bbody,
            (jnp.int32(0), jnp.float32(0.0), jnp.int32(0), jnp.int32(0)))

        def cbody(t, carry):
            cnt_so, sum_so, b_star, found = carry
            o = sb_star * 256 + t * 16
            cv = hcnt[pl.ds(o, 16)]
            new = cnt_so + jnp.sum(cv)
            cross = jnp.logical_and(found == 0, new >= _K)
            b_star = jnp.where(cross, t, b_star)
            found = jnp.where(cross, jnp.int32(1), found)
            take = found == 0
            cnt_so = jnp.where(take, new, cnt_so)
            sum_so = jnp.where(
                take,
                sum_so + jnp.sum(cv.astype(jnp.float32) * _centers(o)),
                sum_so)
            return cnt_so, sum_so, b_star, found

        cnt_so2, sum_so2, b_star, _f2 = lax.fori_loop(
            0, 16, cbody, (cnt_so, sum_so, jnp.int32(0), jnp.int32(0)))

        o = sb_star * 256 + b_star * 16
        cv = hcnt[pl.ds(o, 16)]
        ctr = _centers(o)
        cum = plsc.cumsum(cv) + cnt_so2
        below = cum < _K
        prefix = cum - cv
        onehot = jnp.logical_and(jnp.logical_not(below), prefix < _K)
        cnt_below = cnt_so2 + jnp.sum(jnp.where(below, cv, 0))
        sum_below = sum_so2 + jnp.sum(
            jnp.where(below, cv.astype(jnp.float32) * ctr, zf))
        ctr_bkt = jnp.sum(jnp.where(onehot, ctr, zf))
        r = (_K - cnt_below).astype(jnp.float32)
        obuf[...] = (jnp.full((16,), sum_below, jnp.float32)
                     + jnp.full((16,), r, jnp.float32)
                     * jnp.full((16,), ctr_bkt, jnp.float32))
        pltpu.sync_copy(obuf, out.at[pl.ds(img * 16, 16)])


def kernel(prediction, target, mask):
    p = prediction.reshape(_B, _W, _W)
    t = target.reshape(_B, _W, _W)
    mesh = plsc.VectorSubcoreMesh(core_axis_name="c", subcore_axis_name="s",
                                  num_cores=2, num_subcores=16)
    sums = pl.kernel(
        _sc_body,
        out_type=jax.ShapeDtypeStruct((_B * 16,), jnp.float32),
        mesh=mesh,
        compiler_params=pltpu.CompilerParams(needs_layout_passes=False),
        scratch_types=[
            pltpu.VMEM((_RPC, _W), jnp.float32),
            pltpu.VMEM((_RPC, _W), jnp.float32),
            pltpu.VMEM((_RPC, _W), jnp.float32),
            pltpu.VMEM((_RPC, _W), jnp.float32),
            pltpu.VMEM((_MW,), jnp.int32),
            pltpu.VMEM((_NB,), jnp.int32),
            pltpu.SMEM((_NB // 256,), jnp.int32),
            pltpu.SMEM((_NB // 256,), jnp.float32),
            pltpu.VMEM((16,), jnp.float32),
            pltpu.VMEM_SHARED((8, _NB), jnp.int32),
            pltpu.SemaphoreType.DMA,
            pltpu.SemaphoreType.DMA,
            pltpu.SemaphoreType.DMA,
            pltpu.SemaphoreType.DMA,
        ],
    )(p, t)
    return jnp.mean(sums.reshape(_B, 16)[:, 0]) / (2.0 * _M)


# speedup vs baseline: 1.1124x; 1.0006x over previous
import jax
import jax.numpy as jnp
from jax import lax
from jax.experimental import pallas as pl
from jax.experimental.pallas import tpu as pltpu
from jax.experimental.pallas import tpu_sc as plsc

_B = 16
_W = 512
_M = _W * _W
_K = int(0.8 * _M)
_RPT = 256
_RPC = 32
_NCH = _RPT // _RPC
_SHIFT = 17
_NB = 1 << 14
_MID = 1 << (_SHIFT - 1)
_MW = 2048


def _centers(o):
    bits = ((o + jnp.arange(16, dtype=jnp.int32)) << _SHIFT) | _MID
    return lax.bitcast_convert_type(bits, jnp.float32)


def _sc_body(pred, targ, out, pb0, pb1, tb0, tb1, ctmp, hcnt,
             sbc, sbs, obuf, shc, psem0, psem1, tsem0, tsem1):
    c = lax.axis_index("c")
    s = lax.axis_index("s")
    img = c * 8 + s // 2
    half = s % 2
    slot = s // 2
    row0 = half * _RPT

    zc = jnp.zeros((16,), jnp.int32)
    zf = jnp.zeros((16,), jnp.float32)
    ones = jnp.ones((16,), jnp.int32)

    def start(g, pb, tb, ps, ts):
        pltpu.async_copy(pred.at[img, pl.ds(row0 + g * _RPC, _RPC), :], pb, ps)
        pltpu.async_copy(targ.at[img, pl.ds(row0 + g * _RPC, _RPC), :], tb, ts)

    def wait(pb, tb, ps, ts):
        pltpu.make_async_copy(pred.at[img, pl.ds(row0, _RPC), :], pb, ps).wait()
        pltpu.make_async_copy(targ.at[img, pl.ds(row0, _RPC), :], tb, ts).wait()

    def process(pb, tb):
        @plsc.parallel_loop(0, _W // 16, unroll=1)
        def _proc(k):
            for r in range(_RPC):
                p = pb[r, pl.ds(k * 16, 16)]
                t = tb[r, pl.ds(k * 16, 16)]
                x = jnp.abs(p - t)
                b = lax.shift_right_logical(
                    lax.bitcast_convert_type(x, jnp.int32), _SHIFT)
                plsc.addupdate_scatter(hcnt, [b], ones)

    start(0, pb0, tb0, psem0, tsem0)
    start(1, pb1, tb1, psem1, tsem1)

    @plsc.parallel_loop(0, _NB // 16, unroll=4)
    def _zero(i):
        hcnt[pl.ds(i * 16, 16)] = zc

    def gbody(h, _):
        g = h * 2
        wait(pb0, tb0, psem0, tsem0)

        @pl.when(g + 2 < _NCH)
        def _():
            start(g + 2, pb0, tb0, psem0, tsem0)

        process(pb0, tb0)
        wait(pb1, tb1, psem1, tsem1)

        @pl.when(g + 3 < _NCH)
        def _():
            start(g + 3, pb1, tb1, psem1, tsem1)

        process(pb1, tb1)
        return 0

    lax.fori_loop(0, _NCH // 2, gbody, 0)

    @pl.when(half == 1)
    def _publish():
        pltpu.sync_copy(hcnt, shc.at[slot])

    plsc.subcore_barrier()

    @pl.when(half == 0)
    def _scan():
        for kb in range(_NB // _MW):
            pltpu.sync_copy(shc.at[slot, pl.ds(kb * _MW, _MW)], ctmp)

            @plsc.parallel_loop(0, _MW // 16, unroll=4)
            def _merge(i):
                o = kb * _MW + i * 16
                hcnt[pl.ds(o, 16)] = hcnt[pl.ds(o, 16)] + ctmp[pl.ds(i * 16, 16)]

        @plsc.parallel_loop(0, _NB // 256)
        def _sblk(sb):
            def inner(t, acc):
                o = sb * 256 + t * 16
                cv = hcnt[pl.ds(o, 16)]
                return (acc[0] + cv,
                        acc[1] + cv.astype(jnp.float32) * _centers(o))
            accc, accs = lax.fori_loop(0, 16, inner, (zc, zf), unroll=4)
            sbc[sb] = jnp.sum(accc)
            sbs[sb] = jnp.sum(accs)

        def bbody(j, carry):
            cnt_so, sum_so, sb_star, found = carry
            new = cnt_so + sbc[j]
            cross = jnp.logical_and(found == 0, new >= _K)
            sb_star = jnp.where(cross, j, sb_star)
            found = jnp.where(cross, jnp.int32(1), found)
            take = found == 0
            cnt_so = jnp.where(take, new, cnt_so)
            sum_so = jnp.where(take, sum_so + sbs[j], sum_so)
            return cnt_so, sum_so, sb_star, found

        cnt_so, sum_so, sb_star, _f = lax.fori_loop(
            0, _NB // 256, bbody,
            (jnp.int32(0), jnp.float32(0.0), jnp.int32(0), jnp.int32(0)))

        def cbody(t, carry):
            cnt_so, sum_so, b_star, found = carry
            o = sb_star * 256 + t * 16
            cv = hcnt[pl.ds(o, 16)]
            new = cnt_so + jnp.sum(cv)
            cross = jnp.logical_and(found == 0, new >= _K)
            b_star = jnp.where(cross, t, b_star)
            found = jnp.where(cross, jnp.int32(1), found)
            take = found == 0
            cnt_so = jnp.where(take, new, cnt_so)
            sum_so = jnp.where(
                take,
                sum_so + jnp.sum(cv.astype(jnp.float32) * _centers(o)),
                sum_so)
            return cnt_so, sum_so, b_star, found

        cnt_so2, sum_so2, b_star, _f2 = lax.fori_loop(
            0, 16, cbody, (cnt_so, sum_so, jnp.int32(0), jnp.int32(0)))

        o = sb_star * 256 + b_star * 16
        cv = hcnt[pl.ds(o, 16)]
        ctr = _centers(o)
        cum = plsc.cumsum(cv) + cnt_so2
        below = cum < _K
        prefix = cum - cv
        onehot = jnp.logical_and(jnp.logical_not(below), prefix < _K)
        cnt_below = cnt_so2 + jnp.sum(jnp.where(below, cv, 0))
        sum_below = sum_so2 + jnp.sum(
            jnp.where(below, cv.astype(jnp.float32) * ctr, zf))
        ctr_bkt = jnp.sum(jnp.where(onehot, ctr, zf))
        r = (_K - cnt_below).astype(jnp.float32)
        obuf[...] = (jnp.full((16,), sum_below, jnp.float32)
                     + jnp.full((16,), r, jnp.float32)
                     * jnp.full((16,), ctr_bkt, jnp.float32))
        pltpu.sync_copy(obuf, out.at[pl.ds(img * 16, 16)])


def kernel(prediction, target, mask):
    p = prediction.reshape(_B, _W, _W)
    t = target.reshape(_B, _W, _W)
    mesh = plsc.VectorSubcoreMesh(core_axis_name="c", subcore_axis_name="s",
                                  num_cores=2, num_subcores=16)
    sums = pl.kernel(
        _sc_body,
        out_type=jax.ShapeDtypeStruct((_B * 16,), jnp.float32),
        mesh=mesh,
        compiler_params=pltpu.CompilerParams(needs_layout_passes=False,
                                            disable_bounds_checks=True,
                                            disable_semaphore_checks=True),
        scratch_types=[
            pltpu.VMEM((_RPC, _W), jnp.float32),
            pltpu.VMEM((_RPC, _W), jnp.float32),
            pltpu.VMEM((_RPC, _W), jnp.float32),
            pltpu.VMEM((_RPC, _W), jnp.float32),
            pltpu.VMEM((_MW,), jnp.int32),
            pltpu.VMEM((_NB,), jnp.int32),
            pltpu.SMEM((_NB // 256,), jnp.int32),
            pltpu.SMEM((_NB // 256,), jnp.float32),
            pltpu.VMEM((16,), jnp.float32),
            pltpu.VMEM_SHARED((8, _NB), jnp.int32),
            pltpu.SemaphoreType.DMA,
            pltpu.SemaphoreType.DMA,
            pltpu.SemaphoreType.DMA,
            pltpu.SemaphoreType.DMA,
        ],
    )(p, t)
    return jnp.mean(sums.reshape(_B, 16)[:, 0]) / (2.0 * _M)


# speedup vs baseline: 1.1140x; 1.0015x over previous
import jax
import jax.numpy as jnp
from jax import lax
from jax.experimental import pallas as pl
from jax.experimental.pallas import tpu as pltpu
from jax.experimental.pallas import tpu_sc as plsc

_B = 16
_W = 512
_M = _W * _W
_K = int(0.8 * _M)
_RPT = 256
_RPC = 32
_NCH = _RPT // _RPC
_SHIFT = 17
_NB = 1 << 14
_MID = 1 << (_SHIFT - 1)
_MW = 2048


def _centers(o):
    bits = ((o + jnp.arange(16, dtype=jnp.int32)) << _SHIFT) | _MID
    return lax.bitcast_convert_type(bits, jnp.float32)


def _sc_body(pred, targ, out, pb0, pb1, tb0, tb1, ctmp, hcnt,
             sbc, sbs, obuf, shc, psem0, psem1, tsem0, tsem1):
    c = lax.axis_index("c")
    s = lax.axis_index("s")
    img = c * 8 + s // 2
    half = s % 2
    slot = s // 2
    row0 = half * _RPT

    zc = jnp.zeros((16,), jnp.int32)
    zf = jnp.zeros((16,), jnp.float32)
    ones = jnp.ones((16,), jnp.int32)

    def start(g, pb, tb, ps, ts):
        pltpu.async_copy(pred.at[img, pl.ds(row0 + g * _RPC, _RPC), :], pb, ps)
        pltpu.async_copy(targ.at[img, pl.ds(row0 + g * _RPC, _RPC), :], tb, ts)

    def wait(pb, tb, ps, ts):
        pltpu.make_async_copy(pred.at[img, pl.ds(row0, _RPC), :], pb, ps).wait()
        pltpu.make_async_copy(targ.at[img, pl.ds(row0, _RPC), :], tb, ts).wait()

    def process(pb, tb):
        @plsc.parallel_loop(0, _W // 16, unroll=1)
        def _proc(k):
            for r in range(_RPC):
                p = pb[r, pl.ds(k * 16, 16)]
                t = tb[r, pl.ds(k * 16, 16)]
                x = jnp.abs(p - t)
                b = lax.shift_right_logical(
                    lax.bitcast_convert_type(x, jnp.int32), _SHIFT)
                plsc.addupdate_scatter(hcnt, [b], ones)

    start(0, pb0, tb0, psem0, tsem0)
    start(1, pb1, tb1, psem1, tsem1)

    @plsc.parallel_loop(0, _NB // 16, unroll=4)
    def _zero(i):
        hcnt[pl.ds(i * 16, 16)] = zc

    def gbody(h, _):
        g = h * 2
        wait(pb0, tb0, psem0, tsem0)

        @pl.when(g + 2 < _NCH)
        def _():
            start(g + 2, pb0, tb0, psem0, tsem0)

        process(pb0, tb0)
        wait(pb1, tb1, psem1, tsem1)

        @pl.when(g + 3 < _NCH)
        def _():
            start(g + 3, pb1, tb1, psem1, tsem1)

        process(pb1, tb1)
        return 0

    lax.fori_loop(0, _NCH // 2, gbody, 0)

    @pl.when(half == 1)
    def _publish():
        pltpu.sync_copy(hcnt, shc.at[slot])

    plsc.subcore_barrier()

    @pl.when(half == 0)
    def _scan():
        for kb in range(_NB // _MW):
            pltpu.sync_copy(shc.at[slot, pl.ds(kb * _MW, _MW)], ctmp)

            @plsc.parallel_loop(0, _MW // 16, unroll=4)
            def _merge(i):
                o = kb * _MW + i * 16
                hcnt[pl.ds(o, 16)] = hcnt[pl.ds(o, 16)] + ctmp[pl.ds(i * 16, 16)]

        @plsc.parallel_loop(0, _NB // 256)
        def _sblk(sb):
            def inner(t, acc):
                o = sb * 256 + t * 16
                cv = hcnt[pl.ds(o, 16)]
                return (acc[0] + cv,
                        acc[1] + cv.astype(jnp.float32) * _centers(o))
            accc, accs = lax.fori_loop(0, 16, inner, (zc, zf), unroll=4)
            sbc[sb] = jnp.sum(accc)
            sbs[sb] = jnp.sum(accs)

        def bbody(j, carry):
            cnt_so, sum_so, sb_star, found = carry
            new = cnt_so + sbc[j]
            cross = jnp.logical_and(found == 0, new >= _K)
            sb_star = jnp.where(cross, j, sb_star)
            found = jnp.where(cross, jnp.int32(1), found)
            take = found == 0
            cnt_so = jnp.where(take, new, cnt_so)
            sum_so = jnp.where(take, sum_so + sbs[j], sum_so)
            return cnt_so, sum_so, sb_star, found

        cnt_so, sum_so, sb_star, _f = lax.fori_loop(
            0, _NB // 256, bbody,
            (jnp.int32(0), jnp.float32(0.0), jnp.int32(0), jnp.int32(0)))

        def cbody(t, carry):
            cnt_so, sum_so, b_star, found = carry
            o = sb_star * 256 + t * 16
            cv = hcnt[pl.ds(o, 16)]
            new = cnt_so + jnp.sum(cv)
            cross = jnp.logical_and(found == 0, new >= _K)
            b_star = jnp.where(cross, t, b_star)
            found = jnp.where(cross, jnp.int32(1), found)
            take = found == 0
            cnt_so = jnp.where(take, new, cnt_so)
            sum_so = jnp.where(
                take,
                sum_so + jnp.sum(cv.astype(jnp.float32) * _centers(o)),
                sum_so)
            return cnt_so, sum_so, b_star, found

        cnt_so2, sum_so2, b_star, _f2 = lax.fori_loop(
            0, 16, cbody, (cnt_so, sum_so, jnp.int32(0), jnp.int32(0)))

        o = sb_star * 256 + b_star * 16
        cv = hcnt[pl.ds(o, 16)]
        ctr = _centers(o)
        cum = plsc.cumsum(cv) + cnt_so2
        below = cum < _K
        prefix = cum - cv
        onehot = jnp.logical_and(jnp.logical_not(below), prefix < _K)
        cnt_below = cnt_so2 + jnp.sum(jnp.where(below, cv, 0))
        sum_below = sum_so2 + jnp.sum(
            jnp.where(below, cv.astype(jnp.float32) * ctr, zf))
        ctr_bkt = jnp.sum(jnp.where(onehot, ctr, zf))
        r = (_K - cnt_below).astype(jnp.float32)
        obuf[...] = (jnp.full((16,), sum_below, jnp.float32)
                     + jnp.full((16,), r, jnp.float32)
                     * jnp.full((16,), ctr_bkt, jnp.float32))
        pltpu.sync_copy(obuf, out.at[pl.ds(img * 16, 16)])


def kernel(prediction, target, mask):
    p = prediction.reshape(_B, _W, _W)
    t = target.reshape(_B, _W, _W)
    mesh = plsc.VectorSubcoreMesh(core_axis_name="c", subcore_axis_name="s",
                                  num_cores=2, num_subcores=16)
    sums = pl.kernel(
        _sc_body,
        out_type=jax.ShapeDtypeStruct((_B * 16,), jnp.float32),
        mesh=mesh,
        compiler_params=pltpu.CompilerParams(needs_layout_passes=False),
        scratch_types=[
            pltpu.VMEM((_RPC, _W), jnp.float32),
            pltpu.VMEM((_RPC, _W), jnp.float32),
            pltpu.VMEM((_RPC, _W), jnp.float32),
            pltpu.VMEM((_RPC, _W), jnp.float32),
            pltpu.VMEM((_MW,), jnp.int32),
            pltpu.VMEM((_NB,), jnp.int32),
            pltpu.SMEM((_NB // 256,), jnp.int32),
            pltpu.SMEM((_NB // 256,), jnp.float32),
            pltpu.VMEM((16,), jnp.float32),
            pltpu.VMEM_SHARED((8, _NB), jnp.int32),
            pltpu.SemaphoreType.DMA,
            pltpu.SemaphoreType.DMA,
            pltpu.SemaphoreType.DMA,
            pltpu.SemaphoreType.DMA,
        ],
    )(p, t)
    return jnp.mean(sums.reshape(_B, 16)[:, 0]) / (2.0 * _M)
